# baseline (speedup 1.0000x reference)
import functools
import jax
import jax.numpy as jnp
from jax import lax
from jax.experimental import pallas as pl
from jax.experimental.pallas import tpu as pltpu
from jax.experimental.pallas import tpu_sc as plsc

B = 320000
D = 128
W = D // 2
NC = 2
NS = 16
NW = NC * NS
EPW = B // NW
C = 40
NCHUNK = EPW // C
NBUF = 4
LOOK = 3
MASK = -65536


def _sc_body(pxw_hbm, ii_hbm, jj_hbm, out_hbm, px_sh, ii_v, jj_v, ri, rj, ro,
             gs0, gs1, gs2, gs3, ss0, ss1, ss2, ss3):
    gs = (gs0, gs1, gs2, gs3)
    ss = (ss0, ss1, ss2, ss3)
    sid = lax.axis_index("s")
    wid = sid * NC + lax.axis_index("c")
    base = wid * EPW

    @pl.when(sid == 0)
    def _():
        pltpu.sync_copy(pxw_hbm, px_sh)

    pltpu.sync_copy(ii_hbm.at[pl.ds(base, EPW)], ii_v)
    pltpu.sync_copy(jj_hbm.at[pl.ds(base, EPW)], jj_v)
    plsc.subcore_barrier()

    def start_gather(g, b):
        pltpu.async_copy(px_sh.at[ii_v.at[pl.ds(g * C, C)]], ri.at[b], gs[b])
        pltpu.async_copy(px_sh.at[jj_v.at[pl.ds(g * C, C)]], rj.at[b], gs[b])

    def wait_gather(b):
        pltpu.make_async_copy(px_sh.at[ii_v.at[pl.ds(0, C)]], ri.at[b], gs[b]).wait()
        pltpu.make_async_copy(px_sh.at[jj_v.at[pl.ds(0, C)]], rj.at[b], gs[b]).wait()

    def compute(b):
        def row(r, rc):
            for q in range(W // 16):
                wi = ri[b, r, pl.ds(q * 16, 16)]
                wj = rj[b, r, pl.ds(q * 16, 16)]
                lo = (lax.bitcast_convert_type(wi << 16, jnp.float32)
                      - lax.bitcast_convert_type(wj << 16, jnp.float32))
                hi = (lax.bitcast_convert_type(wi & MASK, jnp.float32)
                      - lax.bitcast_convert_type(wj & MASK, jnp.float32))
                ro[b, r, pl.ds(q * 32, 16)] = lo
                ro[b, r, pl.ds(q * 32 + 16, 16)] = hi
            return rc
        lax.fori_loop(0, C, row, 0, unroll=4)

    def start_store(g, b):
        pltpu.async_copy(ro.at[b], out_hbm.at[pl.ds(base + g * C, C)], ss[b])

    def wait_store(b):
        pltpu.make_async_copy(ro.at[b], out_hbm.at[pl.ds(0, C)], ss[b]).wait()

    for k in range(LOOK):
        start_gather(k, k)

    for g in range(NBUF):
        b = g % NBUF
        wait_gather(b)
        compute(b)
        start_store(g, b)
        start_gather(g + LOOK, (g + LOOK) % NBUF)

    def group(p, carry):
        g0 = p * NBUF
        for b in range(NBUF):
            g = g0 + b
            bq = (b + LOOK) % NBUF

            @pl.when(g < NCHUNK)
            def _():
                wait_gather(b)
                wait_store(b)
                compute(b)
                start_store(g, b)

            @pl.when(g + LOOK < NCHUNK)
            def _():
                start_gather(g + LOOK, bq)
        return carry

    lax.fori_loop(1, (NCHUNK + NBUF - 1) // NBUF, group, 0)

    for b in range(NBUF):
        wait_store(b)


@jax.jit
def _pix_sc(pxw, ii, jj):
    mesh = plsc.VectorSubcoreMesh(core_axis_name="c", subcore_axis_name="s")
    return pl.kernel(
        _sc_body,
        out_type=jax.ShapeDtypeStruct((B, D), jnp.float32),
        mesh=mesh,
        compiler_params=pltpu.CompilerParams(use_tc_tiling_on_sc=False),
        scratch_types=[
            pltpu.VMEM_SHARED((10000, W), jnp.int32),
            pltpu.VMEM((EPW,), jnp.int32),
            pltpu.VMEM((EPW,), jnp.int32),
            pltpu.VMEM((NBUF, C, W), jnp.int32),
            pltpu.VMEM((NBUF, C, W), jnp.int32),
            pltpu.VMEM((NBUF, C, D), jnp.float32),
        ] + [pltpu.SemaphoreType.DMA] * (2 * NBUF),
    )(pxw, ii, jj)


def kernel(px, idx_i, idx_j):
    px_bf = px.astype(jnp.bfloat16)
    px_perm = px_bf.reshape(-1, D // 32, 2, 16).transpose(0, 1, 3, 2)
    pxw = jax.lax.bitcast_convert_type(px_perm.reshape(-1, W, 2), jnp.int32)
    return _pix_sc(pxw, idx_i.astype(jnp.int32), idx_j.astype(jnp.int32))


# speedup vs baseline: 1.4412x; 1.4412x over previous
import functools
import jax
import jax.numpy as jnp
from jax import lax
from jax.experimental import pallas as pl
from jax.experimental.pallas import tpu as pltpu
from jax.experimental.pallas import tpu_sc as plsc

B = 320000
D = 128
NC = 2
NS = 16
NW = NC * NS
EPW = B // NW
C = 80
NCHUNK = EPW // C
NBUF = 5
LOOK = 3


def _sc_body(px_hbm, ii_hbm, jj_hbm, out_hbm, ii_v, jj_v, ri, rj,
             gs0, gs1, gs2, gs3, gs4, ss0, ss1, ss2, ss3, ss4):
    gs = (gs0, gs1, gs2, gs3, gs4)
    ss = (ss0, ss1, ss2, ss3, ss4)
    wid = lax.axis_index("s") * NC + lax.axis_index("c")
    base = wid * EPW

    pltpu.sync_copy(ii_hbm.at[pl.ds(base, EPW)], ii_v)
    pltpu.sync_copy(jj_hbm.at[pl.ds(base, EPW)], jj_v)

    def start_gather(g, b):
        pltpu.async_copy(px_hbm.at[ii_v.at[pl.ds(g * C, C)]], ri.at[b], gs[b])
        pltpu.async_copy(px_hbm.at[jj_v.at[pl.ds(g * C, C)]], rj.at[b], gs[b])

    def wait_gather(b):
        pltpu.make_async_copy(px_hbm.at[ii_v.at[pl.ds(0, C)]], ri.at[b], gs[b]).wait()
        pltpu.make_async_copy(px_hbm.at[jj_v.at[pl.ds(0, C)]], rj.at[b], gs[b]).wait()

    def compute(b):
        def row(r, rc):
            for c8 in range(D // 16):
                sl = pl.ds(c8 * 16, 16)
                plsc.addupdate(ri.at[b, r, sl], -rj[b, r, sl])
            return rc
        lax.fori_loop(0, C, row, 0, unroll=4)

    def start_store(g, b):
        pltpu.async_copy(ri.at[b], out_hbm.at[pl.ds(base + g * C, C)], ss[b])

    def wait_store(b):
        pltpu.make_async_copy(ri.at[b], out_hbm.at[pl.ds(0, C)], ss[b]).wait()

    for k in range(LOOK):
        start_gather(k, k)

    for g in range(NBUF):
        b = g % NBUF
        wait_gather(b)
        compute(b)
        start_store(g, b)
        t = g + LOOK
        bq = t % NBUF
        if t >= NBUF:
            wait_store(bq)
        start_gather(t, bq)

    def group(p, carry):
        g0 = p * NBUF
        for b in range(NBUF):
            g = g0 + b
            bq = (b + LOOK) % NBUF

            @pl.when(g < NCHUNK)
            def _():
                wait_gather(b)
                compute(b)
                start_store(g, b)

            @pl.when(g + LOOK < NCHUNK)
            def _():
                wait_store(bq)
                start_gather(g + LOOK, bq)
        return carry

    lax.fori_loop(1, (NCHUNK + NBUF - 1) // NBUF, group, 0)

    for b in range(NBUF):
        wait_store(b)


@jax.jit
def _pix_sc(px, ii, jj):
    mesh = plsc.VectorSubcoreMesh(core_axis_name="c", subcore_axis_name="s")
    return pl.kernel(
        _sc_body,
        out_type=jax.ShapeDtypeStruct((B, D), jnp.float32),
        mesh=mesh,
        scratch_types=[
            pltpu.VMEM((EPW,), jnp.int32),
            pltpu.VMEM((EPW,), jnp.int32),
            pltpu.VMEM((NBUF, C, D), jnp.float32),
            pltpu.VMEM((NBUF, C, D), jnp.float32),
        ] + [pltpu.SemaphoreType.DMA] * (2 * NBUF),
    )(px, ii, jj)


def kernel(px, idx_i, idx_j):
    return _pix_sc(px, idx_i.astype(jnp.int32), idx_j.astype(jnp.int32))
